# baseline (speedup 1.0000x reference)
import functools

import jax
import jax.numpy as jnp
from jax import lax
from jax.experimental import pallas as pl
from jax.experimental.pallas import tpu as pltpu
from jax.experimental.pallas import tpu_sc as plsc

EMB = 32
ROWS, COLS = 16384, 26
B = ROWS * COLS
CHUNK = 128
NGROUPS = B // CHUNK
BB = ROWS // CHUNK
NC, NS = 2, 16
NW = NC * NS
G_PER_W = NGROUPS // NW
NBUF = 8
NG = 4

VROWS = 1000000
VPAD = 1000064
NVB = VPAD // 128
FULLVB = VROWS // 128
TB_G = FULLVB // NW
TB_NBUF = 8
TB_NG = 4

_mesh = plsc.VectorSubcoreMesh(
    core_axis_name="c", subcore_axis_name="s", num_cores=NC, num_subcores=NS
)


def _diag_vectors():
    lanes = lax.iota(jnp.int32, 16)
    crel = [lax.rem(lanes + k, 16) for k in range(16)]
    return lanes, crel


@functools.partial(
    pl.kernel,
    out_type=jax.ShapeDtypeStruct((NVB, 32, 128), jnp.float32),
    mesh=_mesh,
    scratch_types=[
        pltpu.VMEM((TB_NBUF, 32, 128), jnp.float32),
        pltpu.VMEM((TB_NBUF, 32, 128), jnp.float32),
        pltpu.VMEM((32, 64), jnp.float32),
        pltpu.VMEM((16, 128), jnp.float32),
        pltpu.SemaphoreType.DMA,
        pltpu.SemaphoreType.DMA,
    ],
    compiler_params=pltpu.CompilerParams(use_tc_tiling_on_sc=True,
                                         needs_layout_passes=False),
)
def _table_convert(wt_hbm, out_hbm, sbufs, dbufs, tails, taild, gsem, wsem):
    wid = lax.axis_index("s") * NC + lax.axis_index("c")
    base = wid * TB_G
    lanes, crel = _diag_vectors()
    crel4 = [lax.shift_right_logical(crel[k], 2) for k in range(16)]
    srcc = [cc * 16 + lanes for cc in range(2)]
    d1v = [[(crel[k] & 3) * 32 + cc * 16 + lanes for k in range(16)]
           for cc in range(2)]

    def fire_read(blk, slot):
        pltpu.async_copy(wt_hbm.at[:, pl.ds(blk * 128, 128)], sbufs.at[slot],
                         gsem)

    def wait_read(blk, slot):
        pltpu.make_async_copy(wt_hbm.at[:, pl.ds(blk * 128, 128)],
                              sbufs.at[slot], gsem).wait()

    def transpose_slab(src, dst, nb):
        @pl.loop(0, nb)
        def _b(bb):
            for cc in range(2):
                for k in range(16):
                    blv = bb * 16 + crel[k]
                    v = plsc.load_gather(src, [srcc[cc], blv])
                    plsc.store_scatter(dst, [bb * 4 + crel4[k], d1v[cc][k]], v)

    def fire_write(blk, slot):
        pltpu.async_copy(dbufs.at[slot], out_hbm.at[blk], wsem)

    def wait_write(slot):
        pltpu.make_async_copy(dbufs.at[slot], out_hbm.at[0], wsem).wait()

    for g in range(TB_NG):
        fire_read(base + g, g)

    @pl.loop(0, TB_NBUF)
    def _warm(i):
        s = lax.rem(i, TB_NBUF)
        wait_read(base + i, s)
        transpose_slab(sbufs.at[s], dbufs.at[s], 8)
        fire_write(base + i, s)
        fire_read(base + i + TB_NG, lax.rem(i + TB_NG, TB_NBUF))

    @pl.loop(TB_NBUF, TB_G - TB_NG)
    def _main(i):
        s = lax.rem(i, TB_NBUF)
        wait_write(s)
        wait_read(base + i, s)
        transpose_slab(sbufs.at[s], dbufs.at[s], 8)
        fire_write(base + i, s)
        fire_read(base + i + TB_NG, lax.rem(i + TB_NG, TB_NBUF))

    @pl.loop(TB_G - TB_NG, TB_G)
    def _tailloop(i):
        s = lax.rem(i, TB_NBUF)
        wait_write(s)
        wait_read(base + i, s)
        transpose_slab(sbufs.at[s], dbufs.at[s], 8)
        fire_write(base + i, s)

    for s in range(TB_NBUF):
        wait_write(s)

    @pl.when(wid < FULLVB - NW * TB_G)
    def _extra():
        eb = NW * TB_G + wid
        pltpu.sync_copy(wt_hbm.at[:, pl.ds(eb * 128, 128)], sbufs.at[0])
        transpose_slab(sbufs.at[0], dbufs.at[0], 8)
        pltpu.sync_copy(dbufs.at[0], out_hbm.at[eb])

    @pl.when(wid == 4)
    def _tail():
        for c in range(32):
            pltpu.sync_copy(wt_hbm.at[c, pl.ds(FULLVB * 128, 64)], tails.at[c])
        transpose_slab(tails, taild, 4)
        pltpu.sync_copy(taild, out_hbm.at[FULLVB, pl.ds(0, 16)])


@functools.partial(
    pl.kernel,
    out_type=jax.ShapeDtypeStruct((COLS, EMB // 8, BB, 8, CHUNK), jnp.float32),
    mesh=_mesh,
    scratch_types=[
        pltpu.VMEM((G_PER_W, CHUNK), jnp.int32),
        pltpu.VMEM((NBUF, CHUNK, EMB), jnp.float32),
        pltpu.VMEM((NBUF, EMB // 8, 8, CHUNK), jnp.float32),
        pltpu.SemaphoreType.DMA,
        pltpu.SemaphoreType.DMA,
    ],
    compiler_params=pltpu.CompilerParams(use_tc_tiling_on_sc=False,
                                         needs_layout_passes=False),
)
def _embedding_gather(idx_hbm, table_hbm, out_hbm, idx_v, bufs, tbufs, gsem,
                      wsem):
    wid = lax.axis_index("s") * NC + lax.axis_index("c")
    g0 = wid * G_PER_W
    pltpu.sync_copy(idx_hbm.at[pl.ds(g0, G_PER_W)], idx_v)

    lanes, crel = _diag_vectors()
    row_idx = [blk * 16 + lanes for blk in range(8)]
    col_idx = [[cc * 16 + crel[k] for k in range(16)] for cc in range(2)]
    cb_idx = [[lax.shift_right_logical(col_idx[cc][k], 3) for k in range(16)]
              for cc in range(2)]
    cl_idx = [[lax.rem(col_idx[cc][k], 8) for k in range(16)]
              for cc in range(2)]

    def fire_gather(grp, slot):
        pltpu.async_copy(table_hbm.at[idx_v.at[grp]], bufs.at[slot], gsem)

    def wait_gather(grp, slot):
        pltpu.make_async_copy(table_hbm.at[idx_v.at[grp]], bufs.at[slot],
                              gsem).wait()

    def transpose(slot):
        buf = bufs.at[slot]
        tbuf = tbufs.at[slot]

        @pl.loop(0, 8)
        def _blk(blk):
            row = blk * 16 + lanes
            for cc in range(2):
                for k in range(16):
                    v = plsc.load_gather(buf, [row, col_idx[cc][k]])
                    plsc.store_scatter(
                        tbuf, [cb_idx[cc][k], cl_idx[cc][k], row], v)

    def fire_writes(grp, slot):
        t = lax.div(grp, BB)
        bb = lax.rem(grp, BB)
        for cb in range(EMB // 8):
            pltpu.async_copy(tbufs.at[slot, cb], out_hbm.at[t, cb, bb], wsem)

    def wait_writes(slot):
        for cb in range(EMB // 8):
            pltpu.make_async_copy(tbufs.at[slot, cb], out_hbm.at[0, cb, 0],
                                  wsem).wait()

    for g in range(NG):
        fire_gather(g, g)

    @pl.loop(0, NBUF)
    def _warm(cur):
        s = lax.rem(cur, NBUF)
        wait_gather(cur, s)
        transpose(s)
        fire_writes(g0 + cur, s)
        fire_gather(cur + NG, lax.rem(cur + NG, NBUF))

    @pl.loop(NBUF, G_PER_W - NG)
    def _main(cur):
        s = lax.rem(cur, NBUF)
        wait_writes(s)
        wait_gather(cur, s)
        transpose(s)
        fire_writes(g0 + cur, s)
        fire_gather(cur + NG, lax.rem(cur + NG, NBUF))

    @pl.loop(G_PER_W - NG, G_PER_W)
    def _tail(cur):
        s = lax.rem(cur, NBUF)
        wait_writes(s)
        wait_gather(cur, s)
        transpose(s)
        fire_writes(g0 + cur, s)

    for s in range(NBUF):
        wait_writes(s)


def kernel(input, embed_weight):
    table_lin = _table_convert(embed_weight.T).reshape(VPAD, EMB)
    idx = input.T.reshape(NGROUPS, CHUNK)
    out = _embedding_gather(idx, table_lin)
    return out.transpose(2, 4, 0, 1, 3).reshape(ROWS, COLS, EMB)


# speedup vs baseline: 1.3722x; 1.3722x over previous
import functools

import jax
import jax.numpy as jnp
from jax import lax
from jax.experimental import pallas as pl
from jax.experimental.pallas import tpu as pltpu
from jax.experimental.pallas import tpu_sc as plsc

EMB = 32
ROWS, COLS = 16384, 26
B = ROWS * COLS
CHUNK = 128
NGROUPS = B // CHUNK
BB = ROWS // CHUNK
NC, NS = 2, 16
NW = NC * NS
G_PER_W = NGROUPS // NW
NBUF = 8
NG = 4

VROWS = 1000000
VPAD = 1000064
NVB = VPAD // 128
FULLVB = VROWS // 128
TB_G = FULLVB // NW
TB_NBUF = 8
TB_NG = 4

_mesh = plsc.VectorSubcoreMesh(
    core_axis_name="c", subcore_axis_name="s", num_cores=NC, num_subcores=NS
)


def _diag_vectors():
    lanes = lax.iota(jnp.int32, 16)
    crel = [lax.rem(lanes + k, 16) for k in range(16)]
    return lanes, crel


@functools.partial(
    pl.kernel,
    out_type=jax.ShapeDtypeStruct((NVB, 32, 128), jnp.float32),
    mesh=_mesh,
    scratch_types=[
        pltpu.VMEM((TB_NBUF, 32, 128), jnp.float32),
        pltpu.VMEM((TB_NBUF, 32, 128), jnp.float32),
        pltpu.VMEM((32, 64), jnp.float32),
        pltpu.VMEM((16, 128), jnp.float32),
        pltpu.SemaphoreType.DMA,
        pltpu.SemaphoreType.DMA,
    ],
    compiler_params=pltpu.CompilerParams(use_tc_tiling_on_sc=True,
                                         needs_layout_passes=False),
)
def _table_convert(wt_hbm, out_hbm, sbufs, dbufs, tails, taild, gsem, wsem):
    wid = lax.axis_index("s") * NC + lax.axis_index("c")
    base = wid * TB_G
    lanes, crel = _diag_vectors()
    crel4 = [lax.shift_right_logical(crel[k], 2) for k in range(16)]
    srcc = [cc * 16 + lanes for cc in range(2)]
    d1v = [[(crel[k] & 3) * 32 + cc * 16 + lanes for k in range(16)]
           for cc in range(2)]

    def fire_read(blk, slot):
        for r in range(4):
            pltpu.async_copy(
                wt_hbm.at[pl.ds(r * 8, 8), pl.ds(blk * 128, 128)],
                sbufs.at[slot, pl.ds(r * 8, 8)], gsem)

    def wait_read(blk, slot):
        for r in range(4):
            pltpu.make_async_copy(
                wt_hbm.at[pl.ds(r * 8, 8), pl.ds(blk * 128, 128)],
                sbufs.at[slot, pl.ds(r * 8, 8)], gsem).wait()

    def transpose_slab(src, dst, nb):
        @plsc.parallel_loop(0, nb, 1, unroll=2)
        def _b(bb):
            for cc in range(2):
                for k in range(16):
                    blv = bb * 16 + crel[k]
                    v = plsc.load_gather(src, [srcc[cc], blv])
                    plsc.store_scatter(dst, [bb * 4 + crel4[k], d1v[cc][k]], v)

    def fire_write(blk, slot):
        pltpu.async_copy(dbufs.at[slot], out_hbm.at[blk], wsem)

    def wait_write(slot):
        pltpu.make_async_copy(dbufs.at[slot], out_hbm.at[0], wsem).wait()

    for g in range(TB_NG):
        fire_read(base + g, g)

    @pl.loop(0, TB_NBUF)
    def _warm(i):
        s = lax.rem(i, TB_NBUF)
        wait_read(base + i, s)
        transpose_slab(sbufs.at[s], dbufs.at[s], 8)
        fire_write(base + i, s)
        fire_read(base + i + TB_NG, lax.rem(i + TB_NG, TB_NBUF))

    @pl.loop(TB_NBUF, TB_G - TB_NG)
    def _main(i):
        s = lax.rem(i, TB_NBUF)
        wait_write(s)
        wait_read(base + i, s)
        transpose_slab(sbufs.at[s], dbufs.at[s], 8)
        fire_write(base + i, s)
        fire_read(base + i + TB_NG, lax.rem(i + TB_NG, TB_NBUF))

    @pl.loop(TB_G - TB_NG, TB_G)
    def _tailloop(i):
        s = lax.rem(i, TB_NBUF)
        wait_write(s)
        wait_read(base + i, s)
        transpose_slab(sbufs.at[s], dbufs.at[s], 8)
        fire_write(base + i, s)

    for s in range(TB_NBUF):
        wait_write(s)

    @pl.when(wid < FULLVB - NW * TB_G)
    def _extra():
        eb = NW * TB_G + wid
        pltpu.sync_copy(wt_hbm.at[:, pl.ds(eb * 128, 128)], sbufs.at[0])
        transpose_slab(sbufs.at[0], dbufs.at[0], 8)
        pltpu.sync_copy(dbufs.at[0], out_hbm.at[eb])

    @pl.when(wid == 4)
    def _tail():
        for c in range(32):
            pltpu.sync_copy(wt_hbm.at[c, pl.ds(FULLVB * 128, 64)], tails.at[c])
        transpose_slab(tails, taild, 4)
        pltpu.sync_copy(taild, out_hbm.at[FULLVB, pl.ds(0, 16)])


@functools.partial(
    pl.kernel,
    out_type=jax.ShapeDtypeStruct((COLS, EMB // 8, BB, 8, CHUNK), jnp.float32),
    mesh=_mesh,
    scratch_types=[
        pltpu.VMEM((G_PER_W, CHUNK), jnp.int32),
        pltpu.VMEM((NBUF, CHUNK, EMB), jnp.float32),
        pltpu.VMEM((NBUF, EMB // 8, 8, CHUNK), jnp.float32),
        pltpu.SemaphoreType.DMA,
        pltpu.SemaphoreType.DMA,
    ],
    compiler_params=pltpu.CompilerParams(use_tc_tiling_on_sc=False,
                                         needs_layout_passes=False),
)
def _embedding_gather(idx_hbm, table_hbm, out_hbm, idx_v, bufs, tbufs, gsem,
                      wsem):
    wid = lax.axis_index("s") * NC + lax.axis_index("c")
    g0 = wid * G_PER_W
    pltpu.sync_copy(idx_hbm.at[pl.ds(g0, G_PER_W)], idx_v)

    lanes, crel = _diag_vectors()
    row_idx = [blk * 16 + lanes for blk in range(8)]
    col_idx = [[cc * 16 + crel[k] for k in range(16)] for cc in range(2)]
    cb_idx = [[lax.shift_right_logical(col_idx[cc][k], 3) for k in range(16)]
              for cc in range(2)]
    cl_idx = [[lax.rem(col_idx[cc][k], 8) for k in range(16)]
              for cc in range(2)]

    def fire_gather(grp, slot):
        pltpu.async_copy(table_hbm.at[idx_v.at[grp]], bufs.at[slot], gsem)

    def wait_gather(grp, slot):
        pltpu.make_async_copy(table_hbm.at[idx_v.at[grp]], bufs.at[slot],
                              gsem).wait()

    def transpose(slot):
        buf = bufs.at[slot]
        tbuf = tbufs.at[slot]

        @plsc.parallel_loop(0, 8, 1, unroll=2)
        def _blk(blk):
            row = blk * 16 + lanes
            for cc in range(2):
                for k in range(16):
                    v = plsc.load_gather(buf, [row, col_idx[cc][k]])
                    plsc.store_scatter(
                        tbuf, [cb_idx[cc][k], cl_idx[cc][k], row], v)

    def fire_writes(grp, slot):
        t = lax.div(grp, BB)
        bb = lax.rem(grp, BB)
        for cb in range(EMB // 8):
            pltpu.async_copy(tbufs.at[slot, cb], out_hbm.at[t, cb, bb], wsem)

    def wait_writes(slot):
        for cb in range(EMB // 8):
            pltpu.make_async_copy(tbufs.at[slot, cb], out_hbm.at[0, cb, 0],
                                  wsem).wait()

    for g in range(NG):
        fire_gather(g, g)

    @pl.loop(0, NBUF)
    def _warm(cur):
        s = lax.rem(cur, NBUF)
        wait_gather(cur, s)
        transpose(s)
        fire_writes(g0 + cur, s)
        fire_gather(cur + NG, lax.rem(cur + NG, NBUF))

    @pl.loop(NBUF, G_PER_W - NG)
    def _main(cur):
        s = lax.rem(cur, NBUF)
        wait_writes(s)
        wait_gather(cur, s)
        transpose(s)
        fire_writes(g0 + cur, s)
        fire_gather(cur + NG, lax.rem(cur + NG, NBUF))

    @pl.loop(G_PER_W - NG, G_PER_W)
    def _tail(cur):
        s = lax.rem(cur, NBUF)
        wait_writes(s)
        wait_gather(cur, s)
        transpose(s)
        fire_writes(g0 + cur, s)

    for s in range(NBUF):
        wait_writes(s)


def kernel(input, embed_weight):
    table_lin = _table_convert(embed_weight.T).reshape(VPAD, EMB)
    idx = input.T.reshape(NGROUPS, CHUNK)
    out = _embedding_gather(idx, table_lin)
    return out.transpose(2, 4, 0, 1, 3).reshape(ROWS, COLS, EMB)


# speedup vs baseline: 2.1103x; 1.5379x over previous
import functools

import jax
import jax.numpy as jnp
from jax import lax
from jax.experimental import pallas as pl
from jax.experimental.pallas import tpu as pltpu
from jax.experimental.pallas import tpu_sc as plsc

EMB = 32
ROWS, COLS = 16384, 26
B = ROWS * COLS
CHUNK = 128
NGROUPS = B // CHUNK
BB = ROWS // CHUNK
NC, NS = 2, 16
NW = NC * NS
G_PER_W = NGROUPS // NW
NBUF = 8
NG = 4

VROWS = 1000000
VPAD = 1000064
NVB = VPAD // 128
FULLVB = VROWS // 128
TB_G = FULLVB // NW
TB_NBUF = 8
TB_NG = 4

_mesh = plsc.VectorSubcoreMesh(
    core_axis_name="c", subcore_axis_name="s", num_cores=NC, num_subcores=NS
)


def _diag_vectors():
    lanes = lax.iota(jnp.int32, 16)
    crel = [lax.rem(lanes + k, 16) for k in range(16)]
    return lanes, crel


@functools.partial(
    pl.kernel,
    out_type=jax.ShapeDtypeStruct((NVB, 32, 128), jnp.float32),
    mesh=_mesh,
    scratch_types=[
        pltpu.VMEM((TB_NBUF, 32, 128), jnp.float32),
        pltpu.VMEM((TB_NBUF, 32, 128), jnp.float32),
        pltpu.VMEM((32, 64), jnp.float32),
        pltpu.VMEM((16, 128), jnp.float32),
        pltpu.SemaphoreType.DMA,
        pltpu.SemaphoreType.DMA,
    ],
    compiler_params=pltpu.CompilerParams(use_tc_tiling_on_sc=True,
                                         needs_layout_passes=False),
)
def _table_convert(wt_hbm, out_hbm, sbufs, dbufs, tails, taild, gsem, wsem):
    wid = lax.axis_index("s") * NC + lax.axis_index("c")
    base = wid * TB_G
    lanes, crel = _diag_vectors()
    crel4 = [lax.shift_right_logical(crel[k], 2) for k in range(16)]
    srcc = [cc * 16 + lanes for cc in range(2)]
    d1v = [[(crel[k] & 3) * 32 + cc * 16 + lanes for k in range(16)]
           for cc in range(2)]

    def fire_read(blk, slot):
        for r in range(4):
            pltpu.async_copy(
                wt_hbm.at[pl.ds(r * 8, 8), pl.ds(blk * 128, 128)],
                sbufs.at[slot, pl.ds(r * 8, 8)], gsem)

    def wait_read(blk, slot):
        for r in range(4):
            pltpu.make_async_copy(
                wt_hbm.at[pl.ds(r * 8, 8), pl.ds(blk * 128, 128)],
                sbufs.at[slot, pl.ds(r * 8, 8)], gsem).wait()

    def transpose_slab(src, dst, nb):
        @plsc.parallel_loop(0, nb, 1, unroll=4)
        def _b(bb):
            for cc in range(2):
                for k in range(16):
                    blv = bb * 16 + crel[k]
                    v = plsc.load_gather(src, [srcc[cc], blv])
                    plsc.store_scatter(dst, [bb * 4 + crel4[k], d1v[cc][k]], v)

    def fire_write(blk, slot):
        pltpu.async_copy(dbufs.at[slot], out_hbm.at[blk], wsem)

    def wait_write(slot):
        pltpu.make_async_copy(dbufs.at[slot], out_hbm.at[0], wsem).wait()

    for g in range(TB_NG):
        fire_read(base + g, g)

    @pl.loop(0, TB_NBUF)
    def _warm(i):
        s = lax.rem(i, TB_NBUF)
        wait_read(base + i, s)
        transpose_slab(sbufs.at[s], dbufs.at[s], 8)
        fire_write(base + i, s)
        fire_read(base + i + TB_NG, lax.rem(i + TB_NG, TB_NBUF))

    @pl.loop(TB_NBUF, TB_G - TB_NG)
    def _main(i):
        s = lax.rem(i, TB_NBUF)
        wait_write(s)
        wait_read(base + i, s)
        transpose_slab(sbufs.at[s], dbufs.at[s], 8)
        fire_write(base + i, s)
        fire_read(base + i + TB_NG, lax.rem(i + TB_NG, TB_NBUF))

    @pl.loop(TB_G - TB_NG, TB_G)
    def _tailloop(i):
        s = lax.rem(i, TB_NBUF)
        wait_write(s)
        wait_read(base + i, s)
        transpose_slab(sbufs.at[s], dbufs.at[s], 8)
        fire_write(base + i, s)

    for s in range(TB_NBUF):
        wait_write(s)

    @pl.when(wid < FULLVB - NW * TB_G)
    def _extra():
        eb = NW * TB_G + wid
        pltpu.sync_copy(wt_hbm.at[:, pl.ds(eb * 128, 128)], sbufs.at[0])
        transpose_slab(sbufs.at[0], dbufs.at[0], 8)
        pltpu.sync_copy(dbufs.at[0], out_hbm.at[eb])

    @pl.when(wid == 4)
    def _tail():
        for c in range(32):
            pltpu.sync_copy(wt_hbm.at[c, pl.ds(FULLVB * 128, 64)], tails.at[c])
        transpose_slab(tails, taild, 4)
        pltpu.sync_copy(taild, out_hbm.at[FULLVB, pl.ds(0, 16)])


@functools.partial(
    pl.kernel,
    out_type=jax.ShapeDtypeStruct((COLS, EMB // 8, BB, 8, CHUNK), jnp.float32),
    mesh=_mesh,
    scratch_types=[
        pltpu.VMEM((G_PER_W, CHUNK), jnp.int32),
        pltpu.VMEM((NBUF, CHUNK, EMB), jnp.float32),
        pltpu.VMEM((NBUF, EMB // 8, 8, CHUNK), jnp.float32),
        pltpu.SemaphoreType.DMA,
        pltpu.SemaphoreType.DMA,
    ],
    compiler_params=pltpu.CompilerParams(use_tc_tiling_on_sc=False,
                                         needs_layout_passes=False),
)
def _embedding_gather(idx_hbm, table_hbm, out_hbm, idx_v, bufs, tbufs, gsem,
                      wsem):
    wid = lax.axis_index("s") * NC + lax.axis_index("c")
    g0 = wid * G_PER_W
    pltpu.sync_copy(idx_hbm.at[pl.ds(g0, G_PER_W)], idx_v)

    lanes, crel = _diag_vectors()
    row_idx = [blk * 16 + lanes for blk in range(8)]
    col_idx = [[cc * 16 + crel[k] for k in range(16)] for cc in range(2)]
    cb_idx = [[lax.shift_right_logical(col_idx[cc][k], 3) for k in range(16)]
              for cc in range(2)]
    cl_idx = [[lax.rem(col_idx[cc][k], 8) for k in range(16)]
              for cc in range(2)]

    def fire_gather(grp, slot):
        pltpu.async_copy(table_hbm.at[idx_v.at[grp]], bufs.at[slot], gsem)

    def wait_gather(grp, slot):
        pltpu.make_async_copy(table_hbm.at[idx_v.at[grp]], bufs.at[slot],
                              gsem).wait()

    def transpose(slot):
        buf = bufs.at[slot]
        tbuf = tbufs.at[slot]

        @plsc.parallel_loop(0, 8, 1, unroll=4)
        def _blk(blk):
            row = blk * 16 + lanes
            for cc in range(2):
                for k in range(16):
                    v = plsc.load_gather(buf, [row, col_idx[cc][k]])
                    plsc.store_scatter(
                        tbuf, [cb_idx[cc][k], cl_idx[cc][k], row], v)

    def fire_writes(grp, slot):
        t = lax.div(grp, BB)
        bb = lax.rem(grp, BB)
        for cb in range(EMB // 8):
            pltpu.async_copy(tbufs.at[slot, cb], out_hbm.at[t, cb, bb], wsem)

    def wait_writes(slot):
        for cb in range(EMB // 8):
            pltpu.make_async_copy(tbufs.at[slot, cb], out_hbm.at[0, cb, 0],
                                  wsem).wait()

    for g in range(NG):
        fire_gather(g, g)

    @pl.loop(0, NBUF)
    def _warm(cur):
        s = lax.rem(cur, NBUF)
        wait_gather(cur, s)
        transpose(s)
        fire_writes(g0 + cur, s)
        fire_gather(cur + NG, lax.rem(cur + NG, NBUF))

    @pl.loop(NBUF, G_PER_W - NG)
    def _main(cur):
        s = lax.rem(cur, NBUF)
        wait_writes(s)
        wait_gather(cur, s)
        transpose(s)
        fire_writes(g0 + cur, s)
        fire_gather(cur + NG, lax.rem(cur + NG, NBUF))

    @pl.loop(G_PER_W - NG, G_PER_W)
    def _tail(cur):
        s = lax.rem(cur, NBUF)
        wait_writes(s)
        wait_gather(cur, s)
        transpose(s)
        fire_writes(g0 + cur, s)

    for s in range(NBUF):
        wait_writes(s)


def kernel(input, embed_weight):
    table_lin = _table_convert(embed_weight.T).reshape(VPAD, EMB)
    idx = input.T.reshape(NGROUPS, CHUNK)
    out = _embedding_gather(idx, table_lin)
    return out.transpose(2, 4, 0, 1, 3).reshape(ROWS, COLS, EMB)


# speedup vs baseline: 2.2726x; 1.0769x over previous
import functools

import jax
import jax.numpy as jnp
from jax import lax
from jax.experimental import pallas as pl
from jax.experimental.pallas import tpu as pltpu
from jax.experimental.pallas import tpu_sc as plsc

EMB = 32
ROWS, COLS = 16384, 26
B = ROWS * COLS
CHUNK = 128
NGROUPS = B // CHUNK
BB = ROWS // CHUNK
NC, NS = 2, 16
NW = NC * NS
G_PER_W = NGROUPS // NW
NBUF = 8
NG = 6

VROWS = 1000000
VPAD = 1000064
NVB = VPAD // 128
FULLVB = VROWS // 128
TB_G = FULLVB // NW
TB_NBUF = 8
TB_NG = 6

_mesh = plsc.VectorSubcoreMesh(
    core_axis_name="c", subcore_axis_name="s", num_cores=NC, num_subcores=NS
)


def _diag_vectors():
    lanes = lax.iota(jnp.int32, 16)
    crel = [lax.rem(lanes + k, 16) for k in range(16)]
    return lanes, crel


@functools.partial(
    pl.kernel,
    out_type=jax.ShapeDtypeStruct((NVB, 32, 128), jnp.float32),
    mesh=_mesh,
    scratch_types=[
        pltpu.VMEM((TB_NBUF, 32, 128), jnp.float32),
        pltpu.VMEM((TB_NBUF, 32, 128), jnp.float32),
        pltpu.VMEM((32, 64), jnp.float32),
        pltpu.VMEM((16, 128), jnp.float32),
        pltpu.SemaphoreType.DMA,
        pltpu.SemaphoreType.DMA,
    ],
    compiler_params=pltpu.CompilerParams(use_tc_tiling_on_sc=True,
                                         needs_layout_passes=False),
)
def _table_convert(wt_hbm, out_hbm, sbufs, dbufs, tails, taild, gsem, wsem):
    wid = lax.axis_index("s") * NC + lax.axis_index("c")
    base = wid * TB_G
    lanes, crel = _diag_vectors()
    crel4 = [lax.shift_right_logical(crel[k], 2) for k in range(16)]
    srcc = [cc * 16 + lanes for cc in range(2)]
    d1v = [[(crel[k] & 3) * 32 + cc * 16 + lanes for k in range(16)]
           for cc in range(2)]

    def fire_read(blk, slot):
        for r in range(4):
            pltpu.async_copy(
                wt_hbm.at[pl.ds(r * 8, 8), pl.ds(blk * 128, 128)],
                sbufs.at[slot, pl.ds(r * 8, 8)], gsem)

    def wait_read(blk, slot):
        for r in range(4):
            pltpu.make_async_copy(
                wt_hbm.at[pl.ds(r * 8, 8), pl.ds(blk * 128, 128)],
                sbufs.at[slot, pl.ds(r * 8, 8)], gsem).wait()

    def transpose_slab(src, dst, nb):
        @plsc.parallel_loop(0, nb, 1, unroll=4)
        def _b(bb):
            for cc in range(2):
                for k in range(16):
                    blv = bb * 16 + crel[k]
                    v = plsc.load_gather(src, [srcc[cc], blv])
                    plsc.store_scatter(dst, [bb * 4 + crel4[k], d1v[cc][k]], v)

    def fire_write(blk, slot):
        pltpu.async_copy(dbufs.at[slot], out_hbm.at[blk], wsem)

    def wait_write(slot):
        pltpu.make_async_copy(dbufs.at[slot], out_hbm.at[0], wsem).wait()

    for g in range(TB_NG):
        fire_read(base + g, g)

    @pl.loop(0, TB_NBUF)
    def _warm(i):
        s = lax.rem(i, TB_NBUF)
        wait_read(base + i, s)
        transpose_slab(sbufs.at[s], dbufs.at[s], 8)
        fire_write(base + i, s)
        fire_read(base + i + TB_NG, lax.rem(i + TB_NG, TB_NBUF))

    @pl.loop(TB_NBUF, TB_G - TB_NG)
    def _main(i):
        s = lax.rem(i, TB_NBUF)
        wait_write(s)
        wait_read(base + i, s)
        transpose_slab(sbufs.at[s], dbufs.at[s], 8)
        fire_write(base + i, s)
        fire_read(base + i + TB_NG, lax.rem(i + TB_NG, TB_NBUF))

    @pl.loop(TB_G - TB_NG, TB_G)
    def _tailloop(i):
        s = lax.rem(i, TB_NBUF)
        wait_write(s)
        wait_read(base + i, s)
        transpose_slab(sbufs.at[s], dbufs.at[s], 8)
        fire_write(base + i, s)

    for s in range(TB_NBUF):
        wait_write(s)

    @pl.when(wid < FULLVB - NW * TB_G)
    def _extra():
        eb = NW * TB_G + wid
        pltpu.sync_copy(wt_hbm.at[:, pl.ds(eb * 128, 128)], sbufs.at[0])
        transpose_slab(sbufs.at[0], dbufs.at[0], 8)
        pltpu.sync_copy(dbufs.at[0], out_hbm.at[eb])

    @pl.when(wid == 4)
    def _tail():
        for c in range(32):
            pltpu.sync_copy(wt_hbm.at[c, pl.ds(FULLVB * 128, 64)], tails.at[c])
        transpose_slab(tails, taild, 4)
        pltpu.sync_copy(taild, out_hbm.at[FULLVB, pl.ds(0, 16)])


@functools.partial(
    pl.kernel,
    out_type=jax.ShapeDtypeStruct((COLS, EMB // 8, BB, 8, CHUNK), jnp.float32),
    mesh=_mesh,
    scratch_types=[
        pltpu.VMEM((G_PER_W, CHUNK), jnp.int32),
        pltpu.VMEM((NBUF, CHUNK, EMB), jnp.float32),
        pltpu.VMEM((NBUF, EMB // 8, 8, CHUNK), jnp.float32),
        pltpu.SemaphoreType.DMA,
        pltpu.SemaphoreType.DMA,
    ],
    compiler_params=pltpu.CompilerParams(use_tc_tiling_on_sc=False,
                                         needs_layout_passes=False),
)
def _embedding_gather(idx_hbm, table_hbm, out_hbm, idx_v, bufs, tbufs, gsem,
                      wsem):
    wid = lax.axis_index("s") * NC + lax.axis_index("c")
    g0 = wid * G_PER_W
    pltpu.sync_copy(idx_hbm.at[pl.ds(g0, G_PER_W)], idx_v)

    lanes, crel = _diag_vectors()
    row_idx = [blk * 16 + lanes for blk in range(8)]
    col_idx = [[cc * 16 + crel[k] for k in range(16)] for cc in range(2)]
    cb_idx = [[lax.shift_right_logical(col_idx[cc][k], 3) for k in range(16)]
              for cc in range(2)]
    cl_idx = [[lax.rem(col_idx[cc][k], 8) for k in range(16)]
              for cc in range(2)]

    def fire_gather(grp, slot):
        pltpu.async_copy(table_hbm.at[idx_v.at[grp]], bufs.at[slot], gsem)

    def wait_gather(grp, slot):
        pltpu.make_async_copy(table_hbm.at[idx_v.at[grp]], bufs.at[slot],
                              gsem).wait()

    def transpose(slot):
        buf = bufs.at[slot]
        tbuf = tbufs.at[slot]

        @plsc.parallel_loop(0, 8, 1, unroll=4)
        def _blk(blk):
            row = blk * 16 + lanes
            for cc in range(2):
                for k in range(16):
                    v = plsc.load_gather(buf, [row, col_idx[cc][k]])
                    plsc.store_scatter(
                        tbuf, [cb_idx[cc][k], cl_idx[cc][k], row], v)

    def fire_writes(grp, slot):
        t = lax.div(grp, BB)
        bb = lax.rem(grp, BB)
        for cb in range(EMB // 8):
            pltpu.async_copy(tbufs.at[slot, cb], out_hbm.at[t, cb, bb], wsem)

    def wait_writes(slot):
        for cb in range(EMB // 8):
            pltpu.make_async_copy(tbufs.at[slot, cb], out_hbm.at[0, cb, 0],
                                  wsem).wait()

    for g in range(NG):
        fire_gather(g, g)

    @pl.loop(0, NBUF)
    def _warm(cur):
        s = lax.rem(cur, NBUF)
        wait_gather(cur, s)
        transpose(s)
        fire_writes(g0 + cur, s)
        fire_gather(cur + NG, lax.rem(cur + NG, NBUF))

    @pl.loop(NBUF, G_PER_W - NG)
    def _main(cur):
        s = lax.rem(cur, NBUF)
        wait_writes(s)
        wait_gather(cur, s)
        transpose(s)
        fire_writes(g0 + cur, s)
        fire_gather(cur + NG, lax.rem(cur + NG, NBUF))

    @pl.loop(G_PER_W - NG, G_PER_W)
    def _tail(cur):
        s = lax.rem(cur, NBUF)
        wait_writes(s)
        wait_gather(cur, s)
        transpose(s)
        fire_writes(g0 + cur, s)

    for s in range(NBUF):
        wait_writes(s)


def kernel(input, embed_weight):
    table_lin = _table_convert(embed_weight.T).reshape(VPAD, EMB)
    idx = input.T.reshape(NGROUPS, CHUNK)
    out = _embedding_gather(idx, table_lin)
    return out.transpose(2, 4, 0, 1, 3).reshape(ROWS, COLS, EMB)


# speedup vs baseline: 2.3447x; 1.0317x over previous
import functools

import jax
import jax.numpy as jnp
from jax import lax
from jax.experimental import pallas as pl
from jax.experimental.pallas import tpu as pltpu
from jax.experimental.pallas import tpu_sc as plsc

EMB = 32
ROWS, COLS = 16384, 26
B = ROWS * COLS
CHUNK = 128
NGROUPS = B // CHUNK
BB = ROWS // CHUNK
NC, NS = 2, 16
NW = NC * NS
G_PER_W = NGROUPS // NW
NBUF = 8
NG = 7

VROWS = 1000000
VPAD = 1000064
NVB = VPAD // 128
FULLVB = VROWS // 128
TB_G = FULLVB // NW
TB_NBUF = 8
TB_NG = 7

_mesh = plsc.VectorSubcoreMesh(
    core_axis_name="c", subcore_axis_name="s", num_cores=NC, num_subcores=NS
)


def _diag_vectors():
    lanes = lax.iota(jnp.int32, 16)
    crel = [lax.rem(lanes + k, 16) for k in range(16)]
    return lanes, crel


@functools.partial(
    pl.kernel,
    out_type=jax.ShapeDtypeStruct((NVB, 32, 128), jnp.float32),
    mesh=_mesh,
    scratch_types=[
        pltpu.VMEM((TB_NBUF, 32, 128), jnp.float32),
        pltpu.VMEM((TB_NBUF, 32, 128), jnp.float32),
        pltpu.VMEM((32, 64), jnp.float32),
        pltpu.VMEM((16, 128), jnp.float32),
        pltpu.SemaphoreType.DMA,
        pltpu.SemaphoreType.DMA,
    ],
    compiler_params=pltpu.CompilerParams(use_tc_tiling_on_sc=True,
                                         needs_layout_passes=False),
)
def _table_convert(wt_hbm, out_hbm, sbufs, dbufs, tails, taild, gsem, wsem):
    wid = lax.axis_index("s") * NC + lax.axis_index("c")
    base = wid * TB_G
    lanes, crel = _diag_vectors()
    crel4 = [lax.shift_right_logical(crel[k], 2) for k in range(16)]
    srcc = [cc * 16 + lanes for cc in range(2)]
    d1v = [[(crel[k] & 3) * 32 + cc * 16 + lanes for k in range(16)]
           for cc in range(2)]

    def fire_read(blk, slot):
        for r in range(4):
            pltpu.async_copy(
                wt_hbm.at[pl.ds(r * 8, 8), pl.ds(blk * 128, 128)],
                sbufs.at[slot, pl.ds(r * 8, 8)], gsem)

    def wait_read(blk, slot):
        for r in range(4):
            pltpu.make_async_copy(
                wt_hbm.at[pl.ds(r * 8, 8), pl.ds(blk * 128, 128)],
                sbufs.at[slot, pl.ds(r * 8, 8)], gsem).wait()

    def transpose_slab(src, dst, nb):
        @plsc.parallel_loop(0, nb, 1, unroll=4)
        def _b(bb):
            for cc in range(2):
                for k in range(16):
                    blv = bb * 16 + crel[k]
                    v = plsc.load_gather(src, [srcc[cc], blv])
                    plsc.store_scatter(dst, [bb * 4 + crel4[k], d1v[cc][k]], v)

    def fire_write(blk, slot):
        pltpu.async_copy(dbufs.at[slot], out_hbm.at[blk], wsem)

    def wait_write(slot):
        pltpu.make_async_copy(dbufs.at[slot], out_hbm.at[0], wsem).wait()

    for g in range(TB_NG):
        fire_read(base + g, g)

    @pl.loop(0, TB_NBUF)
    def _warm(i):
        s = lax.rem(i, TB_NBUF)
        wait_read(base + i, s)
        transpose_slab(sbufs.at[s], dbufs.at[s], 8)
        fire_write(base + i, s)
        fire_read(base + i + TB_NG, lax.rem(i + TB_NG, TB_NBUF))

    @pl.loop(TB_NBUF, TB_G - TB_NG)
    def _main(i):
        s = lax.rem(i, TB_NBUF)
        wait_write(s)
        wait_read(base + i, s)
        transpose_slab(sbufs.at[s], dbufs.at[s], 8)
        fire_write(base + i, s)
        fire_read(base + i + TB_NG, lax.rem(i + TB_NG, TB_NBUF))

    @pl.loop(TB_G - TB_NG, TB_G)
    def _tailloop(i):
        s = lax.rem(i, TB_NBUF)
        wait_write(s)
        wait_read(base + i, s)
        transpose_slab(sbufs.at[s], dbufs.at[s], 8)
        fire_write(base + i, s)

    for s in range(TB_NBUF):
        wait_write(s)

    @pl.when(wid < FULLVB - NW * TB_G)
    def _extra():
        eb = NW * TB_G + wid
        pltpu.sync_copy(wt_hbm.at[:, pl.ds(eb * 128, 128)], sbufs.at[0])
        transpose_slab(sbufs.at[0], dbufs.at[0], 8)
        pltpu.sync_copy(dbufs.at[0], out_hbm.at[eb])

    @pl.when(wid == 4)
    def _tail():
        for c in range(32):
            pltpu.sync_copy(wt_hbm.at[c, pl.ds(FULLVB * 128, 64)], tails.at[c])
        transpose_slab(tails, taild, 4)
        pltpu.sync_copy(taild, out_hbm.at[FULLVB, pl.ds(0, 16)])


@functools.partial(
    pl.kernel,
    out_type=jax.ShapeDtypeStruct((COLS, EMB // 8, BB, 8, CHUNK), jnp.float32),
    mesh=_mesh,
    scratch_types=[
        pltpu.VMEM((G_PER_W, CHUNK), jnp.int32),
        pltpu.VMEM((NBUF, CHUNK, EMB), jnp.float32),
        pltpu.VMEM((NBUF, EMB // 8, 8, CHUNK), jnp.float32),
        pltpu.SemaphoreType.DMA,
        pltpu.SemaphoreType.DMA,
    ],
    compiler_params=pltpu.CompilerParams(use_tc_tiling_on_sc=False,
                                         needs_layout_passes=False),
)
def _embedding_gather(idx_hbm, table_hbm, out_hbm, idx_v, bufs, tbufs, gsem,
                      wsem):
    wid = lax.axis_index("s") * NC + lax.axis_index("c")
    g0 = wid * G_PER_W
    pltpu.sync_copy(idx_hbm.at[pl.ds(g0, G_PER_W)], idx_v)

    lanes, crel = _diag_vectors()
    row_idx = [blk * 16 + lanes for blk in range(8)]
    col_idx = [[cc * 16 + crel[k] for k in range(16)] for cc in range(2)]
    cb_idx = [[lax.shift_right_logical(col_idx[cc][k], 3) for k in range(16)]
              for cc in range(2)]
    cl_idx = [[lax.rem(col_idx[cc][k], 8) for k in range(16)]
              for cc in range(2)]

    def fire_gather(grp, slot):
        pltpu.async_copy(table_hbm.at[idx_v.at[grp]], bufs.at[slot], gsem)

    def wait_gather(grp, slot):
        pltpu.make_async_copy(table_hbm.at[idx_v.at[grp]], bufs.at[slot],
                              gsem).wait()

    def transpose(slot):
        buf = bufs.at[slot]
        tbuf = tbufs.at[slot]

        @plsc.parallel_loop(0, 8, 1, unroll=4)
        def _blk(blk):
            row = blk * 16 + lanes
            for cc in range(2):
                for k in range(16):
                    v = plsc.load_gather(buf, [row, col_idx[cc][k]])
                    plsc.store_scatter(
                        tbuf, [cb_idx[cc][k], cl_idx[cc][k], row], v)

    def fire_writes(grp, slot):
        t = lax.div(grp, BB)
        bb = lax.rem(grp, BB)
        for cb in range(EMB // 8):
            pltpu.async_copy(tbufs.at[slot, cb], out_hbm.at[t, cb, bb], wsem)

    def wait_writes(slot):
        for cb in range(EMB // 8):
            pltpu.make_async_copy(tbufs.at[slot, cb], out_hbm.at[0, cb, 0],
                                  wsem).wait()

    for g in range(NG):
        fire_gather(g, g)

    @pl.loop(0, NBUF)
    def _warm(cur):
        s = lax.rem(cur, NBUF)
        wait_gather(cur, s)
        transpose(s)
        fire_writes(g0 + cur, s)
        fire_gather(cur + NG, lax.rem(cur + NG, NBUF))

    @pl.loop(NBUF, G_PER_W - NG)
    def _main(cur):
        s = lax.rem(cur, NBUF)
        wait_writes(s)
        wait_gather(cur, s)
        transpose(s)
        fire_writes(g0 + cur, s)
        fire_gather(cur + NG, lax.rem(cur + NG, NBUF))

    @pl.loop(G_PER_W - NG, G_PER_W)
    def _tail(cur):
        s = lax.rem(cur, NBUF)
        wait_writes(s)
        wait_gather(cur, s)
        transpose(s)
        fire_writes(g0 + cur, s)

    for s in range(NBUF):
        wait_writes(s)


def kernel(input, embed_weight):
    table_lin = _table_convert(embed_weight.T).reshape(VPAD, EMB)
    idx = input.T.reshape(NGROUPS, CHUNK)
    out = _embedding_gather(idx, table_lin)
    return out.transpose(2, 4, 0, 1, 3).reshape(ROWS, COLS, EMB)
